# initial kernel scaffold (unmeasured)
import jax
import jax.numpy as jnp
from jax import lax
from jax.experimental import pallas as pl
from jax.experimental.pallas import tpu as pltpu

N_DEV = 16
SQ = 1024
D = 1024
HQ = 8
DH = 128
CHUNK = SQ // N_DEV
SCALE = 0.08838834764831843


def _body(x_ref, wq_ref, wk_ref, wv_ref, wo_ref, cos_ref, sin_ref,
          out_ref, rs_buf, ag_buf, rs_send, rs_recv, ag_send, ag_recv):
    my = lax.axis_index("i")
    right = lax.rem(my + 1, N_DEV)

    f32 = jnp.float32
    bf16 = jnp.bfloat16
    mm = lambda a, b: lax.dot_general(
        a, b, (((1,), (0,)), ((), ())), preferred_element_type=f32)

    xv = x_ref[:, :]
    q = mm(xv, wq_ref[:, :])
    k = mm(xv, wk_ref[:, :])
    v = mm(xv, wv_ref[:, :]).astype(bf16)
    cos = cos_ref[:, :].astype(f32)
    sin = sin_ref[:, :].astype(f32)

    def rope(t):
        parts = []
        for h in range(HQ):
            a = t[:, h * DH: h * DH + DH // 2]
            b = t[:, h * DH + DH // 2: (h + 1) * DH]
            parts.append(-b)
            parts.append(a)
        return t * cos + jnp.concatenate(parts, axis=1) * sin

    qr = rope(q).astype(bf16)
    kr = rope(k).astype(bf16)

    acc = jnp.zeros((SQ, D), dtype=f32)
    for h in range(HQ):
        qh = qr[:, h * DH:(h + 1) * DH]
        kh = kr[:, h * DH:(h + 1) * DH]
        vh = v[:, h * DH:(h + 1) * DH]
        s = lax.dot_general(
            qh, kh, (((1,), (1,)), ((), ())), preferred_element_type=f32)
        s = s * SCALE
        s = s - jnp.max(s, axis=-1, keepdims=True)
        w = jnp.exp(s)
        w = w / jnp.sum(w, axis=-1, keepdims=True)
        ctx_h = mm(w.astype(bf16), vh).astype(bf16)
        acc = acc + mm(ctx_h, wo_ref[h * DH:(h + 1) * DH, :])
    out_ref[:, :] = acc


    for s_ in range(N_DEV - 1):
        c_send = lax.rem(my - s_ + N_DEV, N_DEV)
        rdma = pltpu.make_async_remote_copy(
            src_ref=out_ref.at[pl.ds(c_send * CHUNK, CHUNK), :],
            dst_ref=rs_buf.at[s_],
            send_sem=rs_send.at[s_],
            recv_sem=rs_recv.at[s_],
            device_id=(right,),
            device_id_type=pl.DeviceIdType.MESH,
        )
        rdma.start()
        rdma.wait()
        c_recv = lax.rem(my - s_ - 1 + N_DEV, N_DEV)
        out_ref[pl.ds(c_recv * CHUNK, CHUNK), :] = (
            out_ref[pl.ds(c_recv * CHUNK, CHUNK), :] + rs_buf[s_]
        )

    for t_ in range(N_DEV - 1):
        c_send = lax.rem(my + 1 - t_ + N_DEV, N_DEV)
        rdma = pltpu.make_async_remote_copy(
            src_ref=out_ref.at[pl.ds(c_send * CHUNK, CHUNK), :],
            dst_ref=ag_buf.at[t_],
            send_sem=ag_send.at[t_],
            recv_sem=ag_recv.at[t_],
            device_id=(right,),
            device_id_type=pl.DeviceIdType.MESH,
        )
        rdma.start()
        rdma.wait()
        c_recv = lax.rem(my - t_ + N_DEV, N_DEV)
        out_ref[pl.ds(c_recv * CHUNK, CHUNK), :] = ag_buf[t_]


def kernel(x, Wq, Wk, Wv, Wo):
    bf16 = jnp.bfloat16
    x2 = x.reshape(SQ, D).astype(bf16)

    def perm(w):
        return w.reshape(D, HQ, DH // 2, 2).transpose(0, 1, 3, 2).reshape(D, HQ * DH)

    wq = perm(Wq).astype(bf16)
    wk = perm(Wk).astype(bf16)
    wv = Wv.astype(bf16)
    wo = Wo.astype(bf16)

    inv = 1.0 / (10000.0 ** (jnp.arange(0, DH, 2, dtype=jnp.float32) / DH))
    pos = jnp.arange(SQ, dtype=jnp.float32)[:, None] * inv[None, :]
    cos_h = jnp.concatenate([jnp.cos(pos), jnp.cos(pos)], axis=-1)
    sin_h = jnp.concatenate([jnp.sin(pos), jnp.sin(pos)], axis=-1)
    cos_t = jnp.tile(cos_h, (1, HQ)).astype(bf16)
    sin_t = jnp.tile(sin_h, (1, HQ)).astype(bf16)

    out = pl.pallas_call(
        _body,
        out_shape=jax.ShapeDtypeStruct((SQ, D), jnp.float32),
        in_specs=[pl.BlockSpec(memory_space=pltpu.VMEM)] * 7,
        out_specs=pl.BlockSpec(memory_space=pltpu.VMEM),
        scratch_shapes=[
            pltpu.VMEM((N_DEV - 1, CHUNK, D), jnp.float32),
            pltpu.VMEM((N_DEV - 1, CHUNK, D), jnp.float32),
            pltpu.SemaphoreType.DMA((N_DEV - 1,)),
            pltpu.SemaphoreType.DMA((N_DEV - 1,)),
            pltpu.SemaphoreType.DMA((N_DEV - 1,)),
            pltpu.SemaphoreType.DMA((N_DEV - 1,)),
        ],
        compiler_params=pltpu.CompilerParams(collective_id=0),
    )(x2, wq, wk, wv, wo, cos_t, sin_t)
    return out.reshape(1, SQ, D)


# baseline (device time: 209480 ns/iter reference)
import jax
import jax.numpy as jnp
from jax import lax
from jax.experimental import pallas as pl
from jax.experimental.pallas import tpu as pltpu

N_DEV = 16
SQ = 1024
D = 1024
HQ = 8
DH = 128
CHUNK = SQ // N_DEV
SCALE = 0.08838834764831843


def _body(x_ref, wq_ref, wk_ref, wv_ref, wo_ref, cos_ref, sin_ref,
          out_ref, rs_buf, ag_buf, rs_send, rs_recv, ag_send, ag_recv):
    my = lax.axis_index("i")
    right = lax.rem(my + 1, N_DEV)

    f32 = jnp.float32
    bf16 = jnp.bfloat16
    mm = lambda a, b: lax.dot_general(
        a, b, (((1,), (0,)), ((), ())), preferred_element_type=f32)

    xv = x_ref[:, :]
    q = mm(xv, wq_ref[:, :])
    k = mm(xv, wk_ref[:, :])
    v = mm(xv, wv_ref[:, :]).astype(bf16)
    cos = cos_ref[:, :].astype(f32)
    sin = sin_ref[:, :].astype(f32)

    def rope(t):
        parts = []
        for h in range(HQ):
            a = t[:, h * DH: h * DH + DH // 2]
            b = t[:, h * DH + DH // 2: (h + 1) * DH]
            parts.append(-b)
            parts.append(a)
        return t * cos + jnp.concatenate(parts, axis=1) * sin

    qr = rope(q).astype(bf16)
    kr = rope(k).astype(bf16)

    acc = jnp.zeros((SQ, D), dtype=f32)
    for h in range(HQ):
        qh = qr[:, h * DH:(h + 1) * DH]
        kh = kr[:, h * DH:(h + 1) * DH]
        vh = v[:, h * DH:(h + 1) * DH]
        s = lax.dot_general(
            qh, kh, (((1,), (1,)), ((), ())), preferred_element_type=f32)
        s = s * SCALE
        s = s - jnp.max(s, axis=-1, keepdims=True)
        w = jnp.exp(s)
        w = w / jnp.sum(w, axis=-1, keepdims=True)
        ctx_h = mm(w.astype(bf16), vh).astype(bf16)
        acc = acc + mm(ctx_h, wo_ref[h * DH:(h + 1) * DH, :])
    out_ref[:, :] = acc


    for s_ in range(N_DEV - 1):
        c_send = lax.rem(my - s_ + N_DEV, N_DEV)
        rdma = pltpu.make_async_remote_copy(
            src_ref=out_ref.at[pl.ds(c_send * CHUNK, CHUNK), :],
            dst_ref=rs_buf.at[s_],
            send_sem=rs_send.at[s_],
            recv_sem=rs_recv.at[s_],
            device_id=(right,),
            device_id_type=pl.DeviceIdType.MESH,
        )
        rdma.start()
        rdma.wait()
        c_recv = lax.rem(my - s_ - 1 + N_DEV, N_DEV)
        out_ref[pl.ds(c_recv * CHUNK, CHUNK), :] = (
            out_ref[pl.ds(c_recv * CHUNK, CHUNK), :] + rs_buf[s_]
        )

    for t_ in range(N_DEV - 1):
        c_send = lax.rem(my + 1 - t_ + N_DEV, N_DEV)
        rdma = pltpu.make_async_remote_copy(
            src_ref=out_ref.at[pl.ds(c_send * CHUNK, CHUNK), :],
            dst_ref=ag_buf.at[t_],
            send_sem=ag_send.at[t_],
            recv_sem=ag_recv.at[t_],
            device_id=(right,),
            device_id_type=pl.DeviceIdType.MESH,
        )
        rdma.start()
        rdma.wait()
        c_recv = lax.rem(my - t_ + N_DEV, N_DEV)
        out_ref[pl.ds(c_recv * CHUNK, CHUNK), :] = ag_buf[t_]


def kernel(x, Wq, Wk, Wv, Wo):
    bf16 = jnp.bfloat16
    x2 = x.reshape(SQ, D).astype(bf16)

    def perm(w):
        return w.reshape(D, HQ, DH // 2, 2).transpose(0, 1, 3, 2).reshape(D, HQ * DH)

    wq = perm(Wq).astype(bf16)
    wk = perm(Wk).astype(bf16)
    wv = Wv.astype(bf16)
    wo = Wo.astype(bf16)

    inv = 1.0 / (10000.0 ** (jnp.arange(0, DH, 2, dtype=jnp.float32) / DH))
    pos = jnp.arange(SQ, dtype=jnp.float32)[:, None] * inv[None, :]
    cos_h = jnp.concatenate([jnp.cos(pos), jnp.cos(pos)], axis=-1)
    sin_h = jnp.concatenate([jnp.sin(pos), jnp.sin(pos)], axis=-1)
    cos_t = jnp.tile(cos_h, (1, HQ)).astype(bf16)
    sin_t = jnp.tile(sin_h, (1, HQ)).astype(bf16)

    out = pl.pallas_call(
        _body,
        out_shape=jax.ShapeDtypeStruct((SQ, D), jnp.float32),
        in_specs=[pl.BlockSpec(memory_space=pltpu.VMEM)] * 7,
        out_specs=pl.BlockSpec(memory_space=pltpu.VMEM),
        scratch_shapes=[
            pltpu.VMEM((N_DEV - 1, CHUNK, D), jnp.float32),
            pltpu.VMEM((N_DEV - 1, CHUNK, D), jnp.float32),
            pltpu.SemaphoreType.DMA((N_DEV - 1,)),
            pltpu.SemaphoreType.DMA((N_DEV - 1,)),
            pltpu.SemaphoreType.DMA((N_DEV - 1,)),
            pltpu.SemaphoreType.DMA((N_DEV - 1,)),
        ],
    )(x2, wq, wk, wv, wo, cos_t, sin_t)
    return out.reshape(1, SQ, D)


# device time: 128455 ns/iter; 1.6308x vs baseline; 1.6308x over previous
import jax
import jax.numpy as jnp
from jax import lax
from jax.experimental import pallas as pl
from jax.experimental.pallas import tpu as pltpu

N_DEV = 16
SQ = 1024
D = 1024
HQ = 8
DH = 128
SCALE = 0.08838834764831843

RS_BITS = (0, 2, 1, 3)
AG_BITS = (3, 1, 2, 0)


def _bit(my, bi):
    return lax.bitwise_and(lax.shift_right_logical(my, bi), 1)


def _body(x_ref, wq_ref, wk_ref, wv_ref, wo_ref, cos_ref, sin_ref,
          out_ref, acc_ref, send_buf,
          rs_b0, rs_b1, rs_b2, rs_b3,
          rs_send, rs_recv, ag_send, ag_recv):
    my = lax.axis_index("i")

    f32 = jnp.float32
    bf16 = jnp.bfloat16
    mm = lambda a, b: lax.dot_general(
        a, b, (((1,), (0,)), ((), ())), preferred_element_type=f32)

    xv = x_ref[:, :]
    q = mm(xv, wq_ref[:, :])
    k = mm(xv, wk_ref[:, :])
    v = mm(xv, wv_ref[:, :]).astype(bf16)
    cos = cos_ref[:, :].astype(f32)
    sin = sin_ref[:, :].astype(f32)

    def rope(t):
        parts = []
        for h in range(HQ):
            a = t[:, h * DH: h * DH + DH // 2]
            b = t[:, h * DH + DH // 2: (h + 1) * DH]
            parts.append(-b)
            parts.append(a)
        return t * cos + jnp.concatenate(parts, axis=1) * sin

    qr = rope(q).astype(bf16)
    kr = rope(k).astype(bf16)

    acc = jnp.zeros((SQ, D), dtype=f32)
    for h in range(HQ):
        qh = qr[:, h * DH:(h + 1) * DH]
        kh = kr[:, h * DH:(h + 1) * DH]
        vh = v[:, h * DH:(h + 1) * DH]
        s = lax.dot_general(
            qh, kh, (((1,), (1,)), ((), ())), preferred_element_type=f32)
        s = s * SCALE
        s = s - jnp.max(s, axis=-1, keepdims=True)
        w = jnp.exp(s)
        w = w / jnp.sum(w, axis=-1, keepdims=True)
        ctx_h = mm(w.astype(bf16), vh).astype(bf16)
        acc = acc + mm(ctx_h, wo_ref[h * DH:(h + 1) * DH, :])
    acc_ref[:, :] = acc

    rs_bufs = [rs_b0, rs_b1, rs_b2, rs_b3]

    base = my * 0
    for r, bi in enumerate(RS_BITS):
        half = 512 >> r
        bit = _bit(my, bi)
        partner = lax.bitwise_xor(my, 1 << bi)
        send_base = base + (1 - bit) * half
        keep_base = base + bit * half
        send_buf[pl.ds(0, half), :] = (
            acc_ref[pl.ds(send_base, half), :].astype(bf16))
        rdma = pltpu.make_async_remote_copy(
            src_ref=send_buf.at[pl.ds(0, half), :],
            dst_ref=rs_bufs[r].at[:, :],
            send_sem=rs_send.at[r],
            recv_sem=rs_recv.at[r],
            device_id=(partner,),
            device_id_type=pl.DeviceIdType.MESH,
        )
        rdma.start()
        rdma.wait()
        acc_ref[pl.ds(keep_base, half), :] = (
            acc_ref[pl.ds(keep_base, half), :]
            + rs_bufs[r][:, :].astype(f32))
        base = keep_base

    out_ref[pl.ds(base, 64), :] = acc_ref[pl.ds(base, 64), :].astype(bf16)

    for r, bi in enumerate(AG_BITS):
        size = 64 << r
        bit = _bit(my, bi)
        partner = lax.bitwise_xor(my, 1 << bi)
        rdma = pltpu.make_async_remote_copy(
            src_ref=out_ref.at[pl.ds(base, size), :],
            dst_ref=out_ref.at[pl.ds(base, size), :],
            send_sem=ag_send.at[r],
            recv_sem=ag_recv.at[r],
            device_id=(partner,),
            device_id_type=pl.DeviceIdType.MESH,
        )
        rdma.start()
        rdma.wait()
        base = base - bit * size


def kernel(x, Wq, Wk, Wv, Wo):
    bf16 = jnp.bfloat16
    x2 = x.reshape(SQ, D).astype(bf16)

    def perm(w):
        return w.reshape(D, HQ, DH // 2, 2).transpose(0, 1, 3, 2).reshape(D, HQ * DH)

    wq = perm(Wq).astype(bf16)
    wk = perm(Wk).astype(bf16)
    wv = Wv.astype(bf16)
    wo = Wo.astype(bf16)

    inv = 1.0 / (10000.0 ** (jnp.arange(0, DH, 2, dtype=jnp.float32) / DH))
    pos = jnp.arange(SQ, dtype=jnp.float32)[:, None] * inv[None, :]
    cos_h = jnp.concatenate([jnp.cos(pos), jnp.cos(pos)], axis=-1)
    sin_h = jnp.concatenate([jnp.sin(pos), jnp.sin(pos)], axis=-1)
    cos_t = jnp.tile(cos_h, (1, HQ)).astype(bf16)
    sin_t = jnp.tile(sin_h, (1, HQ)).astype(bf16)

    out = pl.pallas_call(
        _body,
        out_shape=jax.ShapeDtypeStruct((SQ, D), bf16),
        in_specs=[pl.BlockSpec(memory_space=pltpu.VMEM)] * 7,
        out_specs=pl.BlockSpec(memory_space=pltpu.VMEM),
        scratch_shapes=[
            pltpu.VMEM((SQ, D), jnp.float32),
            pltpu.VMEM((512, D), bf16),
            pltpu.VMEM((512, D), bf16),
            pltpu.VMEM((256, D), bf16),
            pltpu.VMEM((128, D), bf16),
            pltpu.VMEM((64, D), bf16),
            pltpu.SemaphoreType.DMA((4,)),
            pltpu.SemaphoreType.DMA((4,)),
            pltpu.SemaphoreType.DMA((4,)),
            pltpu.SemaphoreType.DMA((4,)),
        ],
    )(x2, wq, wk, wv, wo, cos_t, sin_t)
    return out.reshape(1, SQ, D).astype(jnp.float32)


# device time: 112064 ns/iter; 1.8693x vs baseline; 1.1463x over previous
import jax
import jax.numpy as jnp
from jax import lax
from jax.experimental import pallas as pl
from jax.experimental.pallas import tpu as pltpu

N_DEV = 16
SQ = 1024
D = 1024
HQ = 8
DH = 128
SCALE = 0.08838834764831843

RS_BITS = (0, 2, 1, 3)
AG_BITS = (3, 1, 2, 0)


def _bit(my, bi):
    return lax.bitwise_and(lax.shift_right_logical(my, bi), 1)


def _body(x_ref, wq_ref, wk_ref, wv_ref, wo_ref, cos_ref, sin_ref,
          out_ref, acc_ref, send_buf,
          rs_b0, rs_b1, rs_b2, rs_b3,
          rs_send, rs_recv, ag_send, ag_recv):
    my = lax.axis_index("i")

    f32 = jnp.float32
    bf16 = jnp.bfloat16
    mm = lambda a, b: lax.dot_general(
        a, b, (((1,), (0,)), ((), ())), preferred_element_type=f32)

    xv = x_ref[:, :]
    cos = cos_ref[:, :]
    sin = sin_ref[:, :]

    def rope(t, c, s):
        parts = []
        for h in range(HQ):
            a = t[:, h * DH: h * DH + DH // 2]
            b = t[:, h * DH + DH // 2: (h + 1) * DH]
            parts.append(-b)
            parts.append(a)
        return t * c + jnp.concatenate(parts, axis=1) * s

    kr = rope(mm(xv, wk_ref[:, :]).astype(bf16), cos, sin)
    v = mm(xv, wv_ref[:, :]).astype(bf16)

    def partial_rows(row_start):
        xr = x_ref[pl.ds(row_start, 512), :]
        qb = mm(xr, wq_ref[:, :]).astype(bf16)
        cb = cos_ref[pl.ds(row_start, 512), :]
        sb = sin_ref[pl.ds(row_start, 512), :]
        qrb = rope(qb, cb, sb)
        acch = jnp.zeros((512, D), f32)
        for h in range(HQ):
            hs = slice(h * DH, (h + 1) * DH)
            s = lax.dot_general(
                qrb[:, hs], kr[:, hs], (((1,), (1,)), ((), ())),
                preferred_element_type=f32)
            w = jnp.exp(s * SCALE)
            w = w / jnp.sum(w, axis=-1, keepdims=True)
            ctx = mm(w.astype(bf16), v[:, hs]).astype(bf16)
            acch = acch + mm(ctx, wo_ref[hs, :])
        return acch

    rs_bufs = [rs_b0, rs_b1, rs_b2, rs_b3]

    bit0 = _bit(my, RS_BITS[0])
    send_base = (1 - bit0) * 512
    keep_base = bit0 * 512

    acc_a = partial_rows(send_base)
    send_buf[pl.ds(0, 512), :] = acc_a.astype(bf16)
    rdma0 = pltpu.make_async_remote_copy(
        src_ref=send_buf.at[pl.ds(0, 512), :],
        dst_ref=rs_b0.at[:, :],
        send_sem=rs_send.at[0],
        recv_sem=rs_recv.at[0],
        device_id=(lax.bitwise_xor(my, 1 << RS_BITS[0]),),
        device_id_type=pl.DeviceIdType.MESH,
    )
    rdma0.start()
    acc_b = partial_rows(keep_base)
    acc_ref[pl.ds(keep_base, 512), :] = acc_b
    rdma0.wait()

    base = keep_base
    prev = rs_b0
    for r in range(1, 4):
        half = 512 >> r
        bi = RS_BITS[r]
        bit = _bit(my, bi)
        send_start = base + (1 - bit) * half
        keep_start = base + bit * half
        sendv = (acc_ref[pl.ds(send_start, half), :]
                 + prev[pl.ds((1 - bit) * half, half), :].astype(f32))
        send_buf[pl.ds(0, half), :] = sendv.astype(bf16)
        rdma = pltpu.make_async_remote_copy(
            src_ref=send_buf.at[pl.ds(0, half), :],
            dst_ref=rs_bufs[r].at[:, :],
            send_sem=rs_send.at[r],
            recv_sem=rs_recv.at[r],
            device_id=(lax.bitwise_xor(my, 1 << bi),),
            device_id_type=pl.DeviceIdType.MESH,
        )
        rdma.start()
        acc_ref[pl.ds(keep_start, half), :] = (
            acc_ref[pl.ds(keep_start, half), :]
            + prev[pl.ds(bit * half, half), :].astype(f32))
        rdma.wait()
        base = keep_start
        prev = rs_bufs[r]

    own = acc_ref[pl.ds(base, 64), :] + rs_b3[:, :].astype(f32)
    out_ref[pl.ds(base, 64), :] = own.astype(bf16)

    for r, bi in enumerate(AG_BITS):
        size = 64 << r
        bit = _bit(my, bi)
        rdma = pltpu.make_async_remote_copy(
            src_ref=out_ref.at[pl.ds(base, size), :],
            dst_ref=out_ref.at[pl.ds(base, size), :],
            send_sem=ag_send.at[r],
            recv_sem=ag_recv.at[r],
            device_id=(lax.bitwise_xor(my, 1 << bi),),
            device_id_type=pl.DeviceIdType.MESH,
        )
        rdma.start()
        rdma.wait()
        base = base - bit * size


def kernel(x, Wq, Wk, Wv, Wo):
    bf16 = jnp.bfloat16
    x2 = x.reshape(SQ, D).astype(bf16)

    def perm(w):
        return w.reshape(D, HQ, DH // 2, 2).transpose(0, 1, 3, 2).reshape(D, HQ * DH)

    wq = perm(Wq).astype(bf16)
    wk = perm(Wk).astype(bf16)
    wv = Wv.astype(bf16)
    wo = Wo.astype(bf16)

    inv = 1.0 / (10000.0 ** (jnp.arange(0, DH, 2, dtype=jnp.float32) / DH))
    pos = jnp.arange(SQ, dtype=jnp.float32)[:, None] * inv[None, :]
    cos_h = jnp.concatenate([jnp.cos(pos), jnp.cos(pos)], axis=-1)
    sin_h = jnp.concatenate([jnp.sin(pos), jnp.sin(pos)], axis=-1)
    cos_t = jnp.tile(cos_h, (1, HQ)).astype(bf16)
    sin_t = jnp.tile(sin_h, (1, HQ)).astype(bf16)

    out = pl.pallas_call(
        _body,
        out_shape=jax.ShapeDtypeStruct((SQ, D), bf16),
        in_specs=[pl.BlockSpec(memory_space=pltpu.VMEM)] * 7,
        out_specs=pl.BlockSpec(memory_space=pltpu.VMEM),
        scratch_shapes=[
            pltpu.VMEM((SQ, D), jnp.float32),
            pltpu.VMEM((512, D), bf16),
            pltpu.VMEM((512, D), bf16),
            pltpu.VMEM((256, D), bf16),
            pltpu.VMEM((128, D), bf16),
            pltpu.VMEM((64, D), bf16),
            pltpu.SemaphoreType.DMA((4,)),
            pltpu.SemaphoreType.DMA((4,)),
            pltpu.SemaphoreType.DMA((4,)),
            pltpu.SemaphoreType.DMA((4,)),
        ],
    )(x2, wq, wk, wv, wo, cos_t, sin_t)
    return out.reshape(1, SQ, D).astype(jnp.float32)
